# Initial kernel scaffold; baseline (speedup 1.0000x reference)
import jax
import jax.numpy as jnp
from jax.experimental import pallas as pl


def kernel(x, edge_index, batch, proteins, W1, a_src1, a_dst1, b1, W2, a_src2, a_dst2, b2, fc_g1_w, fc_g1_b, emb, conv_w, conv_b, fc_xt1_w, fc_xt1_b, fc1_w, fc1_b, fc2_w, fc2_b, out_w, out_b):
    raise NotImplementedError("write your pallas kernel here")


# speedup vs baseline: 1.1173x; 1.1173x over previous
import functools

import jax
import jax.numpy as jnp
from jax import lax
from jax.experimental import pallas as pl
from jax.experimental.pallas import tpu as pltpu


def _gat_conv_fast(x, src, dst, W, a_src, a_dst, b, heads, out_ch):
    N = x.shape[0]
    h = (x @ W).reshape(N, heads, out_ch)
    alpha_src = jnp.sum(h * a_src[None, :, :], axis=-1)
    alpha_dst = jnp.sum(h * a_dst[None, :, :], axis=-1)
    e = jax.nn.leaky_relu(alpha_src[src] + alpha_dst[dst], 0.2)
    ee = jnp.exp(e)
    den = jax.ops.segment_sum(ee, dst, num_segments=N)
    num = jax.ops.segment_sum(h[src] * ee[:, :, None], dst, num_segments=N)
    out = num / (den[:, :, None] + 1e-16)
    return out.reshape(N, heads * out_ch) + b


def _tail_body(g_ref, xt_ref, fc1_w_ref, fc1_b_ref, fc2_w_ref, fc2_b_ref,
               out_w_ref, out_b_ref, o_ref):
    xc = jnp.concatenate([g_ref[...], xt_ref[...]], axis=1)
    h1 = jnp.maximum(
        jnp.dot(xc, fc1_w_ref[...], preferred_element_type=jnp.float32)
        + fc1_b_ref[...][None, :], 0.0)
    h2 = jnp.maximum(
        jnp.dot(h1, fc2_w_ref[...], preferred_element_type=jnp.float32)
        + fc2_b_ref[...][None, :], 0.0)
    o_ref[...] = (
        jnp.dot(h2, out_w_ref[...], preferred_element_type=jnp.float32)
        + out_b_ref[...][None, :])


def _tail(g, xt, fc1_w, fc1_b, fc2_w, fc2_b, out_w, out_b):
    B = g.shape[0]
    return pl.pallas_call(
        _tail_body,
        out_shape=jax.ShapeDtypeStruct((B, 1), jnp.float32),
    )(g, xt, fc1_w, fc1_b, fc2_w, fc2_b, out_w, out_b)


def kernel(x, edge_index, batch, proteins, W1, a_src1, a_dst1, b1, W2, a_src2,
           a_dst2, b2, fc_g1_w, fc_g1_b, emb, conv_w, conv_b, fc_xt1_w,
           fc_xt1_b, fc1_w, fc1_b, fc2_w, fc2_b, out_w, out_b):
    N = x.shape[0]
    B = proteins.shape[0]
    loop = jnp.arange(N, dtype=edge_index.dtype)
    src = jnp.concatenate([edge_index[0], loop])
    dst = jnp.concatenate([edge_index[1], loop])

    h = jax.nn.elu(_gat_conv_fast(x, src, dst, W1, a_src1, a_dst1, b1, 10, 78))
    h = jax.nn.relu(_gat_conv_fast(h, src, dst, W2, a_src2, a_dst2, b2, 1, 128))
    g = jax.ops.segment_max(h, batch, num_segments=B)
    g = jax.nn.relu(g @ fc_g1_w + fc_g1_b)

    e_xt = emb[proteins]
    conv = lax.conv_general_dilated(
        e_xt, conv_w, window_strides=(1,), padding='VALID',
        dimension_numbers=('NCH', 'OIH', 'NCH'))
    conv = jax.nn.relu(conv + conv_b[None, :, None])
    xt = conv.reshape(B, 32 * 121) @ fc_xt1_w + fc_xt1_b

    return _tail(g, xt, fc1_w, fc1_b, fc2_w, fc2_b, out_w, out_b)


# speedup vs baseline: 4.2760x; 3.8270x over previous
import functools

import jax
import jax.numpy as jnp
from jax import lax
from jax.experimental import pallas as pl
from jax.experimental.pallas import tpu as pltpu
from jax.experimental.pallas import tpu_sc as plsc

N = 10000
BATCH = 256
NT = 16
NC = 2
NHEADS = 10
HPC = NHEADS // NC
D1 = 80
D2 = 144
RPT = 632
NROWS = RPT * NT
DSTPAD = 16000
CAP = 12288
KE = 128
SCAN = 2048
ETOT = 170000
EPAD = 172032

_GDN = lax.GatherDimensionNumbers(
    offset_dims=(), collapsed_slice_dims=(0,), start_index_map=(0,))


def _lane(v, l):
    idx = jnp.full((16, 1), l, jnp.int32)
    return lax.gather(v, idx, _GDN, (1,),
                      mode=lax.GatherScatterMode.PROMISE_IN_BOUNDS)


def _conv1_body(src_hbm, dst_hbm, asrc_hbm, adst_hbm, ht_hbm, zrow_hbm,
                out_hbm, bsrc_out, bdst_out, cnt_out,
                csrc_v, cdst_v, bsrc_v, bdst_v, idx_h,
                arow_v, brow_v, rows_v, acc_v, cntv_v, sem1, sem2, sem3):
    c = lax.axis_index("c")
    s = lax.axis_index("s")
    lo = s * RPT
    iota = lax.iota(jnp.int32, 16)

    def initb(i, _):
        bsrc_v[pl.ds(i * 16, 16)] = jnp.zeros((16,), jnp.int32)
        bdst_v[pl.ds(i * 16, 16)] = jnp.broadcast_to(lo, (16,))
        return 0
    lax.fori_loop(0, CAP // 16, initb, 0)

    def scan_chunk(ci, cnt):
        pltpu.sync_copy(src_hbm.at[pl.ds(ci * SCAN, SCAN)], csrc_v)
        pltpu.sync_copy(dst_hbm.at[pl.ds(ci * SCAN, SCAN)], cdst_v)

        def scan_vreg(j, cnt):
            sv = csrc_v[pl.ds(j * 16, 16)]
            dv = cdst_v[pl.ds(j * 16, 16)]
            m = (dv >= lo) & (dv < lo + RPT)
            mi = m.astype(jnp.int32)
            pos = jnp.minimum(cnt + plsc.cumsum(mi) - mi, CAP - 1)
            plsc.store_scatter(bsrc_v, [pos], sv, mask=m)
            plsc.store_scatter(bdst_v, [pos], dv, mask=m)
            return jnp.minimum(cnt + jnp.sum(mi), CAP - 16)
        return lax.fori_loop(0, SCAN // 16, scan_vreg, cnt)
    cnt = lax.fori_loop(0, EPAD // SCAN, scan_chunk, jnp.int32(0))

    cntv_v[...] = jnp.broadcast_to(cnt, (16,))

    @pl.when(c == 0)
    def _publish():
        pltpu.sync_copy(bsrc_v, bsrc_out.at[pl.ds(s * CAP, CAP)])
        pltpu.sync_copy(bdst_v, bdst_out.at[pl.ds(s * CAP, CAP)])
        pltpu.sync_copy(cntv_v, cnt_out.at[pl.ds(s * 16, 16)])

    def head_pass(k, _):
        h = c * HPC + k

        pltpu.sync_copy(zrow_hbm, acc_v)

        hsplat = jnp.broadcast_to(h, (16,))

        def chunk_body(ci, _, h=h, hsplat=hsplat):
            base = ci * KE
            for j in range(KE // 16):
                sv = bsrc_v[pl.ds(base + j * 16, 16)]
                idx_h[pl.ds(j * 16, 16)] = sv * NHEADS + h
            d1 = pltpu.async_copy(ht_hbm.at[idx_h], rows_v, sem1)
            d2 = pltpu.async_copy(
                asrc_hbm.at[bsrc_v.at[pl.ds(base, KE)]], arow_v, sem2)
            d3 = pltpu.async_copy(
                adst_hbm.at[bdst_v.at[pl.ds(base, KE)]], brow_v, sem3)
            d1.wait()
            d2.wait()
            d3.wait()
            for j in range(KE // 16):
                eidx = base + j * 16 + iota
                av = plsc.load_gather(arow_v, [j * 16 + iota, hsplat])
                bv = plsc.load_gather(brow_v, [j * 16 + iota, hsplat])
                ev = av + bv
                ev = jnp.where(ev >= 0.0, ev, 0.2 * ev)
                ee = jnp.exp(ev)
                ee = jnp.where(eidx < cnt, ee, 0.0)
                dloc = bdst_v[pl.ds(base + j * 16, 16)] - lo
                for l in range(16):
                    se = _lane(ee, l)
                    sd = _lane(dloc, l)
                    for cc in range(5):
                        hv = rows_v[j * 16 + l, pl.ds(cc * 16, 16)]
                        plsc.addupdate_scatter(
                            acc_v, [sd, cc * 16 + iota], hv * se)
            return 0
        lax.fori_loop(0, CAP // KE, chunk_body, 0)
        pltpu.sync_copy(acc_v, out_hbm.at[h, pl.ds(lo, RPT)])
        return 0
    lax.fori_loop(0, HPC, head_pass, 0)


def _conv2_body(bsrc_hbm, bdst_hbm, cnt_hbm, asrc_hbm, adst_hbm, ht_hbm,
                zrow_hbm, out_hbm,
                sidx_v, didx_v, rows_v, arow_v, brow_v, acc_v, cntv_v,
                sem1, sem2, sem3):
    c = lax.axis_index("c")
    s = lax.axis_index("s")
    lo = s * RPT
    iota = lax.iota(jnp.int32, 16)
    half = CAP // 2
    estart = c * half

    pltpu.sync_copy(cnt_hbm.at[pl.ds(s * 16, 16)], cntv_v)
    pltpu.sync_copy(zrow_hbm, acc_v)
    cntv = cntv_v[...]

    def chunk_body(ci, _):
        start = estart + ci * KE
        pltpu.sync_copy(bsrc_hbm.at[pl.ds(s * CAP + start, KE)], sidx_v)
        pltpu.sync_copy(bdst_hbm.at[pl.ds(s * CAP + start, KE)], didx_v)
        d1 = pltpu.async_copy(ht_hbm.at[sidx_v], rows_v, sem1)
        d2 = pltpu.async_copy(asrc_hbm.at[sidx_v], arow_v, sem2)
        d3 = pltpu.async_copy(adst_hbm.at[didx_v], brow_v, sem3)
        d1.wait()
        d2.wait()
        d3.wait()
        zsplat = jnp.zeros((16,), jnp.int32)
        for j in range(KE // 16):
            eidx = start + j * 16 + iota
            av = plsc.load_gather(arow_v, [j * 16 + iota, zsplat])
            bv = plsc.load_gather(brow_v, [j * 16 + iota, zsplat])
            ev = av + bv
            ev = jnp.where(ev >= 0.0, ev, 0.2 * ev)
            ee = jnp.exp(ev)
            ee = jnp.where(eidx < cntv, ee, 0.0)
            dloc = didx_v[pl.ds(j * 16, 16)] - lo
            for l in range(16):
                se = _lane(ee, l)
                sd = _lane(dloc, l)
                for cc in range(D2 // 16):
                    hv = rows_v[j * 16 + l, pl.ds(cc * 16, 16)]
                    plsc.addupdate_scatter(
                        acc_v, [sd, cc * 16 + iota], hv * se)
        return 0
    lax.fori_loop(0, half // KE, chunk_body, 0)
    pltpu.sync_copy(acc_v, out_hbm.at[c, pl.ds(lo, RPT)])


_MESH = plsc.VectorSubcoreMesh(core_axis_name="c", subcore_axis_name="s")

_conv1_call = pl.kernel(
    _conv1_body,
    out_type=(
        jax.ShapeDtypeStruct((NHEADS, NROWS, D1), jnp.float32),
        jax.ShapeDtypeStruct((NT * CAP,), jnp.int32),
        jax.ShapeDtypeStruct((NT * CAP,), jnp.int32),
        jax.ShapeDtypeStruct((NT * 16,), jnp.int32),
    ),
    mesh=_MESH,
    compiler_params=pltpu.CompilerParams(needs_layout_passes=False, use_tc_tiling_on_sc=False),
    scratch_types=[
        pltpu.VMEM((SCAN,), jnp.int32),
        pltpu.VMEM((SCAN,), jnp.int32),
        pltpu.VMEM((CAP,), jnp.int32),
        pltpu.VMEM((CAP,), jnp.int32),
        pltpu.VMEM((KE,), jnp.int32),
        pltpu.VMEM((KE, 16), jnp.float32),
        pltpu.VMEM((KE, 16), jnp.float32),
        pltpu.VMEM((KE, D1), jnp.float32),
        pltpu.VMEM((RPT, D1), jnp.float32),
        pltpu.VMEM((16,), jnp.int32),
        pltpu.SemaphoreType.DMA,
        pltpu.SemaphoreType.DMA,
        pltpu.SemaphoreType.DMA,
    ],
)

_conv2_call = pl.kernel(
    _conv2_body,
    out_type=jax.ShapeDtypeStruct((NC, NROWS, D2), jnp.float32),
    mesh=_MESH,
    compiler_params=pltpu.CompilerParams(needs_layout_passes=False, use_tc_tiling_on_sc=False),
    scratch_types=[
        pltpu.VMEM((KE,), jnp.int32),
        pltpu.VMEM((KE,), jnp.int32),
        pltpu.VMEM((KE, D2), jnp.float32),
        pltpu.VMEM((KE, 16), jnp.float32),
        pltpu.VMEM((KE, 16), jnp.float32),
        pltpu.VMEM((RPT, D2), jnp.float32),
        pltpu.VMEM((16,), jnp.int32),
        pltpu.SemaphoreType.DMA,
        pltpu.SemaphoreType.DMA,
        pltpu.SemaphoreType.DMA,
    ],
)


def _tail_body(g_ref, xt_ref, fc1_w_ref, fc1_b_ref, fc2_w_ref, fc2_b_ref,
               out_w_ref, out_b_ref, o_ref):
    xc = jnp.concatenate([g_ref[...], xt_ref[...]], axis=1)
    h1 = jnp.maximum(
        jnp.dot(xc, fc1_w_ref[...], preferred_element_type=jnp.float32)
        + fc1_b_ref[...][None, :], 0.0)
    h2 = jnp.maximum(
        jnp.dot(h1, fc2_w_ref[...], preferred_element_type=jnp.float32)
        + fc2_b_ref[...][None, :], 0.0)
    o_ref[...] = (
        jnp.dot(h2, out_w_ref[...], preferred_element_type=jnp.float32)
        + out_b_ref[...][None, :])


def _tail(g, xt, fc1_w, fc1_b, fc2_w, fc2_b, out_w, out_b):
    return pl.pallas_call(
        _tail_body,
        out_shape=jax.ShapeDtypeStruct((g.shape[0], 1), jnp.float32),
    )(g, xt, fc1_w, fc1_b, fc2_w, fc2_b, out_w, out_b)


def kernel(x, edge_index, batch, proteins, W1, a_src1, a_dst1, b1, W2, a_src2,
           a_dst2, b2, fc_g1_w, fc_g1_b, emb, conv_w, conv_b, fc_xt1_w,
           fc_xt1_b, fc1_w, fc1_b, fc2_w, fc2_b, out_w, out_b):
    loop = jnp.arange(N, dtype=edge_index.dtype)
    npad = EPAD - ETOT
    srcp = jnp.concatenate(
        [edge_index[0], loop, jnp.zeros((npad,), jnp.int32)])
    dstp = jnp.concatenate(
        [edge_index[1], loop, jnp.full((npad,), DSTPAD, jnp.int32)])

    h1 = x @ W1
    hr = h1.reshape(N, NHEADS, 78)
    alpha_src = jnp.einsum('nhc,hc->nh', hr, a_src1)
    alpha_dst = jnp.einsum('nhc,hc->nh', hr, a_dst1)
    asrc_tab = jnp.pad(alpha_src, ((0, 0), (0, 6)))
    adst_tab = jnp.pad(alpha_dst, ((0, 0), (0, 6)))
    ht1 = jnp.concatenate(
        [hr, jnp.ones((N, NHEADS, 1), jnp.float32),
         jnp.zeros((N, NHEADS, 1), jnp.float32)], axis=-1
    ).reshape(N * NHEADS, D1)
    zrow1 = jnp.zeros((RPT, D1), jnp.float32)

    out1, bsrc, bdst, cnts = _conv1_call(
        srcp, dstp, asrc_tab, adst_tab, ht1, zrow1)

    num1 = out1[:, :N, :78].transpose(1, 0, 2)
    den1 = out1[:, :N, 78].T[:, :, None]
    gat1 = (num1 / (den1 + 1e-16)).reshape(N, NHEADS * 78) + b1
    h2in = jax.nn.elu(gat1)

    h2 = h2in @ W2
    asrc2 = jnp.pad((h2 @ a_src2[0])[:, None], ((0, 0), (0, 15)))
    adst2 = jnp.pad((h2 @ a_dst2[0])[:, None], ((0, 0), (0, 15)))
    ht2 = jnp.concatenate(
        [h2, jnp.ones((N, 1), jnp.float32),
         jnp.zeros((N, D2 - 129), jnp.float32)], axis=1)
    zrow2 = jnp.zeros((RPT, D2), jnp.float32)

    out2 = _conv2_call(bsrc, bdst, cnts, asrc2, adst2, ht2, zrow2)
    o2 = out2[0, :N] + out2[1, :N]
    h3 = jax.nn.relu(o2[:, :128] / (o2[:, 128:129] + 1e-16) + b2)

    g = jax.ops.segment_max(h3, batch, num_segments=BATCH)
    g = jax.nn.relu(g @ fc_g1_w + fc_g1_b)

    e_xt = emb[proteins]
    conv = lax.conv_general_dilated(
        e_xt, conv_w, window_strides=(1,), padding='VALID',
        dimension_numbers=('NCH', 'OIH', 'NCH'))
    conv = jax.nn.relu(conv + conv_b[None, :, None])
    xt = conv.reshape(BATCH, 32 * 121) @ fc_xt1_w + fc_xt1_b

    return _tail(g, xt, fc1_w, fc1_b, fc2_w, fc2_b, out_w, out_b)


# speedup vs baseline: 5.2228x; 1.2214x over previous
import functools

import jax
import jax.numpy as jnp
from jax import lax
from jax.experimental import pallas as pl
from jax.experimental.pallas import tpu as pltpu
from jax.experimental.pallas import tpu_sc as plsc

N = 10000
BATCH = 256
NT = 16
NC = 2
NHEADS = 10
HPC = NHEADS // NC
D1 = 80
D2 = 144
RPT = 632
NROWS = RPT * NT
DSTPAD = 16000
CAP = 12288
KE = 192
SCAN = 2048
ETOT = 170000
EPAD = 172032

_GDN = lax.GatherDimensionNumbers(
    offset_dims=(), collapsed_slice_dims=(0,), start_index_map=(0,))


def _lane(v, l):
    idx = jnp.full((16, 1), l, jnp.int32)
    return lax.gather(v, idx, _GDN, (1,),
                      mode=lax.GatherScatterMode.PROMISE_IN_BOUNDS)


def _conv1_body(src_hbm, dst_hbm, adst_hbm, ht_hbm, zrow_hbm,
                out_hbm, bsrc_out, bdst_out, cnt_out,
                csrc_v, cdst_v, bsrc_v, bdst_v,
                idxh0, idxh1, rows0, rows1, adst_loc,
                acc_v, cntv_v, semh0, semh1):
    c = lax.axis_index("c")
    s = lax.axis_index("s")
    lo = s * RPT
    iota = lax.iota(jnp.int32, 16)

    def initb(i, _):
        bsrc_v[pl.ds(i * 16, 16)] = jnp.zeros((16,), jnp.int32)
        bdst_v[pl.ds(i * 16, 16)] = jnp.broadcast_to(lo, (16,))
        return 0
    lax.fori_loop(0, CAP // 16, initb, 0)

    def scan_chunk(ci, cnt):
        pltpu.sync_copy(src_hbm.at[pl.ds(ci * SCAN, SCAN)], csrc_v)
        pltpu.sync_copy(dst_hbm.at[pl.ds(ci * SCAN, SCAN)], cdst_v)

        def scan_vreg(j, cnt):
            sv = csrc_v[pl.ds(j * 16, 16)]
            dv = cdst_v[pl.ds(j * 16, 16)]
            m = (dv >= lo) & (dv < lo + RPT)
            mi = m.astype(jnp.int32)
            pos = jnp.minimum(cnt + plsc.cumsum(mi) - mi, CAP - 1)
            plsc.store_scatter(bsrc_v, [pos], sv, mask=m)
            plsc.store_scatter(bdst_v, [pos], dv, mask=m)
            return jnp.minimum(cnt + jnp.sum(mi), CAP - 16)
        return lax.fori_loop(0, SCAN // 16, scan_vreg, cnt)
    cnt = lax.fori_loop(0, EPAD // SCAN, scan_chunk, jnp.int32(0))

    cntv_v[...] = jnp.broadcast_to(cnt, (16,))

    @pl.when(c == 0)
    def _publish():
        pltpu.sync_copy(bsrc_v, bsrc_out.at[pl.ds(s * CAP, CAP)])
        pltpu.sync_copy(bdst_v, bdst_out.at[pl.ds(s * CAP, CAP)])
        pltpu.sync_copy(cntv_v, cnt_out.at[pl.ds(s * 16, 16)])

    nchunks = CAP // KE
    slots = ((idxh0, rows0, semh0), (idxh1, rows1, semh1))
    col79 = jnp.full((16,), 79, jnp.int32)

    pltpu.sync_copy(adst_hbm.at[pl.ds(lo, RPT)], adst_loc)

    def head_pass(k, _):
        h = c * HPC + k

        pltpu.sync_copy(zrow_hbm, acc_v)

        hsplat = jnp.broadcast_to(h, (16,))

        def issue(ci, slot):
            idxh, rows, semh = slots[slot]
            base = jnp.minimum(ci, nchunks - 1) * KE
            for j in range(KE // 16):
                sv = bsrc_v[pl.ds(base + j * 16, 16)]
                idxh[pl.ds(j * 16, 16)] = sv * NHEADS + h
            pltpu.async_copy(ht_hbm.at[idxh], rows, semh)

        def wait(slot):
            idxh, rows, semh = slots[slot]
            pltpu.make_async_copy(ht_hbm.at[idxh], rows, semh).wait()

        def compute(ci, slot):
            idxh, rows, semh = slots[slot]
            base = ci * KE
            for j in range(KE // 16):
                eidx = base + j * 16 + iota
                dloc16 = bdst_v[pl.ds(base + j * 16, 16)] - lo
                av = plsc.load_gather(rows, [j * 16 + iota, col79])
                bv = plsc.load_gather(adst_loc, [dloc16, hsplat])
                ev = av + bv
                ev = jnp.where(ev >= 0.0, ev, 0.2 * ev)
                ee = jnp.exp(ev)
                ee = jnp.where(eidx < cnt, ee, 0.0)

                @plsc.parallel_loop(0, 16, 1, unroll=8)
                def lp(l, j=j, ee=ee, dloc=dloc16, rows=rows):
                    se = _lane(ee, l)
                    sd = _lane(dloc, l)
                    for cc in range(5):
                        hv = rows[j * 16 + l, pl.ds(cc * 16, 16)]
                        plsc.addupdate_scatter(
                            acc_v, [sd, cc * 16 + iota], hv * se)

        issue(jnp.int32(0), 0)

        def pair_body(p, _):
            ci0 = 2 * p
            issue(ci0 + 1, 1)
            wait(0)
            compute(ci0, 0)
            issue(ci0 + 2, 0)
            wait(1)
            compute(ci0 + 1, 1)
            return 0
        lax.fori_loop(0, nchunks // 2, pair_body, 0)
        wait(0)
        pltpu.sync_copy(acc_v, out_hbm.at[h, pl.ds(lo, RPT)])
        return 0
    lax.fori_loop(0, HPC, head_pass, 0)


KE2 = 128


def _conv2_body(bsrc_hbm, bdst_hbm, cnt_hbm, adst_hbm, ht_hbm,
                zrow_hbm, out_hbm,
                sidx0, sidx1, didx0, didx1, rows0, rows1, adst_loc,
                acc_v, cntv_v, semh0, semh1):
    c = lax.axis_index("c")
    s = lax.axis_index("s")
    lo = s * RPT
    iota = lax.iota(jnp.int32, 16)
    half = CAP // 2
    estart = c * half
    nchunks = half // KE2

    pltpu.sync_copy(cnt_hbm.at[pl.ds(s * 16, 16)], cntv_v)
    pltpu.sync_copy(zrow_hbm, acc_v)
    pltpu.sync_copy(adst_hbm.at[pl.ds(lo, RPT)], adst_loc)
    cntv = cntv_v[...]
    zsplat = jnp.zeros((16,), jnp.int32)
    col129 = jnp.full((16,), 129, jnp.int32)

    slots = ((sidx0, didx0, rows0, semh0), (sidx1, didx1, rows1, semh1))

    def issue(ci, slot):
        sidx, didx, rows, semh = slots[slot]
        start = estart + jnp.minimum(ci, nchunks - 1) * KE2
        pltpu.sync_copy(bsrc_hbm.at[pl.ds(s * CAP + start, KE2)], sidx)
        pltpu.sync_copy(bdst_hbm.at[pl.ds(s * CAP + start, KE2)], didx)
        pltpu.async_copy(ht_hbm.at[sidx], rows, semh)

    def wait(slot):
        sidx, didx, rows, semh = slots[slot]
        pltpu.make_async_copy(ht_hbm.at[sidx], rows, semh).wait()

    def compute(ci, slot):
        sidx, didx, rows, semh = slots[slot]
        start = estart + ci * KE2
        for j in range(KE2 // 16):
            eidx = start + j * 16 + iota
            dloc = didx[pl.ds(j * 16, 16)] - lo
            av = plsc.load_gather(rows, [j * 16 + iota, col129])
            bv = plsc.load_gather(adst_loc, [dloc])
            ev = av + bv
            ev = jnp.where(ev >= 0.0, ev, 0.2 * ev)
            ee = jnp.exp(ev)
            ee = jnp.where(eidx < cntv, ee, 0.0)

            @plsc.parallel_loop(0, 16, 1, unroll=8)
            def lp(l, j=j, ee=ee, dloc=dloc, rows=rows):
                se = _lane(ee, l)
                sd = _lane(dloc, l)
                for cc in range(D2 // 16):
                    hv = rows[j * 16 + l, pl.ds(cc * 16, 16)]
                    plsc.addupdate_scatter(
                        acc_v, [sd, cc * 16 + iota], hv * se)

    issue(jnp.int32(0), 0)

    def pair_body(p, _):
        ci0 = 2 * p
        issue(ci0 + 1, 1)
        wait(0)
        compute(ci0, 0)
        issue(ci0 + 2, 0)
        wait(1)
        compute(ci0 + 1, 1)
        return 0
    lax.fori_loop(0, nchunks // 2, pair_body, 0)
    wait(0)
    pltpu.sync_copy(acc_v, out_hbm.at[c, pl.ds(lo, RPT)])


_MESH = plsc.VectorSubcoreMesh(core_axis_name="c", subcore_axis_name="s")

_conv1_call = pl.kernel(
    _conv1_body,
    out_type=(
        jax.ShapeDtypeStruct((NHEADS, NROWS, D1), jnp.float32),
        jax.ShapeDtypeStruct((NT * CAP,), jnp.int32),
        jax.ShapeDtypeStruct((NT * CAP,), jnp.int32),
        jax.ShapeDtypeStruct((NT * 16,), jnp.int32),
    ),
    mesh=_MESH,
    compiler_params=pltpu.CompilerParams(needs_layout_passes=False, use_tc_tiling_on_sc=False),
    scratch_types=[
        pltpu.VMEM((SCAN,), jnp.int32),
        pltpu.VMEM((SCAN,), jnp.int32),
        pltpu.VMEM((CAP,), jnp.int32),
        pltpu.VMEM((CAP,), jnp.int32),
        pltpu.VMEM((KE,), jnp.int32),
        pltpu.VMEM((KE,), jnp.int32),
        pltpu.VMEM((KE, D1), jnp.float32),
        pltpu.VMEM((KE, D1), jnp.float32),
        pltpu.VMEM((RPT, 16), jnp.float32),
        pltpu.VMEM((RPT, D1), jnp.float32),
        pltpu.VMEM((16,), jnp.int32),
        pltpu.SemaphoreType.DMA,
        pltpu.SemaphoreType.DMA,
    ],
)

_conv2_call = pl.kernel(
    _conv2_body,
    out_type=jax.ShapeDtypeStruct((NC, NROWS, D2), jnp.float32),
    mesh=_MESH,
    compiler_params=pltpu.CompilerParams(needs_layout_passes=False, use_tc_tiling_on_sc=False),
    scratch_types=[
        pltpu.VMEM((KE2,), jnp.int32),
        pltpu.VMEM((KE2,), jnp.int32),
        pltpu.VMEM((KE2,), jnp.int32),
        pltpu.VMEM((KE2,), jnp.int32),
        pltpu.VMEM((KE2, D2), jnp.float32),
        pltpu.VMEM((KE2, D2), jnp.float32),
        pltpu.VMEM((RPT,), jnp.float32),
        pltpu.VMEM((RPT, D2), jnp.float32),
        pltpu.VMEM((16,), jnp.int32),
        pltpu.SemaphoreType.DMA,
        pltpu.SemaphoreType.DMA,
    ],
)


def _tail_body(g_ref, xt_ref, fc1_w_ref, fc1_b_ref, fc2_w_ref, fc2_b_ref,
               out_w_ref, out_b_ref, o_ref):
    xc = jnp.concatenate([g_ref[...], xt_ref[...]], axis=1)
    h1 = jnp.maximum(
        jnp.dot(xc, fc1_w_ref[...], preferred_element_type=jnp.float32)
        + fc1_b_ref[...][None, :], 0.0)
    h2 = jnp.maximum(
        jnp.dot(h1, fc2_w_ref[...], preferred_element_type=jnp.float32)
        + fc2_b_ref[...][None, :], 0.0)
    o_ref[...] = (
        jnp.dot(h2, out_w_ref[...], preferred_element_type=jnp.float32)
        + out_b_ref[...][None, :])


def _tail(g, xt, fc1_w, fc1_b, fc2_w, fc2_b, out_w, out_b):
    return pl.pallas_call(
        _tail_body,
        out_shape=jax.ShapeDtypeStruct((g.shape[0], 1), jnp.float32),
    )(g, xt, fc1_w, fc1_b, fc2_w, fc2_b, out_w, out_b)


def kernel(x, edge_index, batch, proteins, W1, a_src1, a_dst1, b1, W2, a_src2,
           a_dst2, b2, fc_g1_w, fc_g1_b, emb, conv_w, conv_b, fc_xt1_w,
           fc_xt1_b, fc1_w, fc1_b, fc2_w, fc2_b, out_w, out_b):
    loop = jnp.arange(N, dtype=edge_index.dtype)
    npad = EPAD - ETOT
    srcp = jnp.concatenate(
        [edge_index[0], loop, jnp.zeros((npad,), jnp.int32)])
    dstp = jnp.concatenate(
        [edge_index[1], loop, jnp.full((npad,), DSTPAD, jnp.int32)])

    h1 = x @ W1
    hr = h1.reshape(N, NHEADS, 78)
    alpha_src = jnp.einsum('nhc,hc->nh', hr, a_src1)
    alpha_dst = jnp.einsum('nhc,hc->nh', hr, a_dst1)
    adst_tab = jnp.pad(alpha_dst, ((0, 0), (0, 6)))
    adst_tab = jnp.pad(adst_tab, ((0, NROWS - N), (0, 0)))
    ht1 = jnp.concatenate(
        [hr, jnp.ones((N, NHEADS, 1), jnp.float32),
         alpha_src[:, :, None]], axis=-1
    ).reshape(N * NHEADS, D1)
    zrow1 = jnp.zeros((RPT, D1), jnp.float32)

    out1, bsrc, bdst, cnts = _conv1_call(
        srcp, dstp, adst_tab, ht1, zrow1)

    num1 = out1[:, :N, :78].transpose(1, 0, 2)
    den1 = out1[:, :N, 78].T[:, :, None]
    gat1 = (num1 / (den1 + 1e-16)).reshape(N, NHEADS * 78) + b1
    h2in = jax.nn.elu(gat1)

    h2 = h2in @ W2
    asrc2 = h2 @ a_src2[0]
    adst2 = jnp.pad(h2 @ a_dst2[0], (0, NROWS - N))
    ht2 = jnp.concatenate(
        [h2, jnp.ones((N, 1), jnp.float32), asrc2[:, None],
         jnp.zeros((N, D2 - 130), jnp.float32)], axis=1)
    zrow2 = jnp.zeros((RPT, D2), jnp.float32)

    out2 = _conv2_call(bsrc, bdst, cnts, adst2, ht2, zrow2)
    o2 = out2[0, :N] + out2[1, :N]
    h3 = jax.nn.relu(o2[:, :128] / (o2[:, 128:129] + 1e-16) + b2)

    g = jax.ops.segment_max(h3, batch, num_segments=BATCH)
    g = jax.nn.relu(g @ fc_g1_w + fc_g1_b)

    e_xt = emb[proteins]
    conv = lax.conv_general_dilated(
        e_xt, conv_w, window_strides=(1,), padding='VALID',
        dimension_numbers=('NCH', 'OIH', 'NCH'))
    conv = jax.nn.relu(conv + conv_b[None, :, None])
    xt = conv.reshape(BATCH, 32 * 121) @ fc_xt1_w + fc_xt1_b

    return _tail(g, xt, fc1_w, fc1_b, fc2_w, fc2_b, out_w, out_b)


# speedup vs baseline: 8.6321x; 1.6528x over previous
import functools

import jax
import jax.numpy as jnp
from jax import lax
from jax.experimental import pallas as pl
from jax.experimental.pallas import tpu as pltpu
from jax.experimental.pallas import tpu_sc as plsc

N = 10000
BATCH = 256
NT = 16
NC = 2
NHEADS = 10
HPC = NHEADS // NC
D1 = 80
D2 = 144
RPT = 632
NROWS = RPT * NT
DSTPAD = 16000
CAP = 12288
SCAN = 2048
ETOT = 170000
EPAD = 172032
W = 512
NSB = 20
NWIN = NSB * W
KE2 = 128

_GDN = lax.GatherDimensionNumbers(
    offset_dims=(), collapsed_slice_dims=(0,), start_index_map=(0,))


def _lane(v, l):
    idx = jnp.full((16, 1), l, jnp.int32)
    return lax.gather(v, idx, _GDN, (1,),
                      mode=lax.GatherScatterMode.PROMISE_IN_BOUNDS)


def _conv1_body(src_hbm, dst_hbm, adst_hbm, ht_hbm, zrow_hbm,
                out_hbm, bkey_out, cnt_out,
                csrc_v, cdst_v, bkey_v, bkey2_v, win_v, adst_loc,
                acc_v, cntv_v, sboff_s, semw):
    c = lax.axis_index("c")
    s = lax.axis_index("s")
    lo = s * RPT
    iota = lax.iota(jnp.int32, 16)

    def initb(i, _):
        bkey_v[pl.ds(i * 16, 16)] = jnp.zeros((16,), jnp.int32)
        bkey2_v[pl.ds(i * 16, 16)] = jnp.zeros((16,), jnp.int32)
        return 0
    lax.fori_loop(0, CAP // 16, initb, 0)

    def scan_chunk(ci, cnt):
        pltpu.sync_copy(src_hbm.at[pl.ds(ci * SCAN, SCAN)], csrc_v)
        pltpu.sync_copy(dst_hbm.at[pl.ds(ci * SCAN, SCAN)], cdst_v)

        def scan_vreg(j, cnt):
            sv = csrc_v[pl.ds(j * 16, 16)]
            dv = cdst_v[pl.ds(j * 16, 16)]
            m = (dv >= lo) & (dv < lo + RPT)
            mi = m.astype(jnp.int32)
            pos = jnp.minimum(cnt + plsc.cumsum(mi) - mi, CAP - 1)
            plsc.store_scatter(bkey_v, [pos], sv * 1024 + (dv - lo), mask=m)
            return jnp.minimum(cnt + jnp.sum(mi), CAP - 16)
        return lax.fori_loop(0, SCAN // 16, scan_vreg, cnt)
    cnt = lax.fori_loop(0, EPAD // SCAN, scan_chunk, jnp.int32(0))

    nv = (cnt + 15) // 16

    def part_sb(sb, off):
        sboff_s[sb] = off

        def part_vreg(i, off):
            kv = bkey_v[pl.ds(i * 16, 16)]
            valid = (i * 16 + iota) < cnt
            m = (lax.shift_right_logical(kv, 19) == sb) & valid
            mi = m.astype(jnp.int32)
            tot = jnp.sum(mi)

            @pl.when(tot > 0)
            def _():
                pos = jnp.minimum(off + plsc.cumsum(mi) - mi, CAP - 1)
                plsc.store_scatter(bkey2_v, [pos], kv, mask=m)
            return off + tot
        return lax.fori_loop(0, nv, part_vreg, off)
    total = lax.fori_loop(0, NSB, part_sb, jnp.int32(0))
    sboff_s[NSB] = total

    cntv_v[...] = jnp.broadcast_to(cnt, (16,))

    @pl.when(c == 0)
    def _publish():
        pltpu.sync_copy(bkey2_v, bkey_out.at[pl.ds(s * CAP, CAP)])
        pltpu.sync_copy(cntv_v, cnt_out.at[pl.ds(s * 16, 16)])

    col79 = jnp.full((16,), 79, jnp.int32)

    def head_pass(k, _):
        h = c * HPC + k
        pltpu.sync_copy(adst_hbm.at[h, pl.ds(lo, RPT)], adst_loc)
        pltpu.sync_copy(zrow_hbm, acc_v)

        def sb_pass(sb, _):
            pltpu.async_copy(
                ht_hbm.at[h, pl.ds(sb * W, W)], win_v, semw).wait()
            sbstart = sboff_s[sb]
            sbcnt = sboff_s[sb + 1] - sbstart
            wbase = sb * W

            def edge_vreg(i, _):
                off = sbstart + i * 16
                kv = bkey2_v[pl.ds(off, 16)]
                srcw = jnp.clip(
                    lax.shift_right_logical(kv, 10) - wbase, 0, W - 1)
                dloc = kv & 1023
                av = plsc.load_gather(win_v, [srcw, col79])
                bv = plsc.load_gather(adst_loc, [dloc])
                ev = av + bv
                ev = jnp.where(ev >= 0.0, ev, 0.2 * ev)
                ee = jnp.exp(ev)
                ee = jnp.where(i * 16 + iota < sbcnt, ee, 0.0)

                @plsc.parallel_loop(0, 16, 1, unroll=8)
                def lp(l, ee=ee, dloc=dloc, srcw=srcw):
                    se = _lane(ee, l)
                    sd = _lane(dloc, l)
                    sw = _lane(srcw, l)
                    for cc in range(5):
                        hv = plsc.load_gather(win_v, [sw, cc * 16 + iota])
                        plsc.addupdate_scatter(
                            acc_v, [sd, cc * 16 + iota], hv * se)
                return 0
            lax.fori_loop(0, (sbcnt + 15) // 16, edge_vreg, 0)
            return 0
        lax.fori_loop(0, NSB, sb_pass, 0)
        pltpu.sync_copy(acc_v, out_hbm.at[h, pl.ds(lo, RPT)])
        return 0
    lax.fori_loop(0, HPC, head_pass, 0)


def _conv2_body(bkey_hbm, cnt_hbm, adst_hbm, ht_hbm, zrow_hbm, out_hbm,
                kbuf0, kbuf1, sidx0, sidx1, rows0, rows1, adst_loc,
                acc_v, cntv_v, semh0, semh1):
    c = lax.axis_index("c")
    s = lax.axis_index("s")
    lo = s * RPT
    iota = lax.iota(jnp.int32, 16)
    half = CAP // 2
    estart = c * half
    nchunks = half // KE2

    pltpu.sync_copy(cnt_hbm.at[pl.ds(s * 16, 16)], cntv_v)
    pltpu.sync_copy(zrow_hbm, acc_v)
    pltpu.sync_copy(adst_hbm.at[pl.ds(lo, RPT)], adst_loc)
    cntv = cntv_v[...]
    col129 = jnp.full((16,), 129, jnp.int32)

    slots = ((kbuf0, sidx0, rows0, semh0), (kbuf1, sidx1, rows1, semh1))

    def issue(ci, slot):
        kbuf, sidx, rows, semh = slots[slot]
        start = estart + jnp.minimum(ci, nchunks - 1) * KE2
        pltpu.sync_copy(bkey_hbm.at[pl.ds(s * CAP + start, KE2)], kbuf)
        for j in range(KE2 // 16):
            kv = kbuf[pl.ds(j * 16, 16)]
            sidx[pl.ds(j * 16, 16)] = lax.shift_right_logical(kv, 10)
        pltpu.async_copy(ht_hbm.at[sidx], rows, semh)

    def wait(slot):
        kbuf, sidx, rows, semh = slots[slot]
        pltpu.make_async_copy(ht_hbm.at[sidx], rows, semh).wait()

    def compute(ci, slot):
        kbuf, sidx, rows, semh = slots[slot]
        start = estart + ci * KE2
        for j in range(KE2 // 16):
            eidx = start + j * 16 + iota
            dloc = kbuf[pl.ds(j * 16, 16)] & 1023
            av = plsc.load_gather(rows, [j * 16 + iota, col129])
            bv = plsc.load_gather(adst_loc, [dloc])
            ev = av + bv
            ev = jnp.where(ev >= 0.0, ev, 0.2 * ev)
            ee = jnp.exp(ev)
            ee = jnp.where(eidx < cntv, ee, 0.0)

            @plsc.parallel_loop(0, 16, 1, unroll=8)
            def lp(l, j=j, ee=ee, dloc=dloc, rows=rows):
                se = _lane(ee, l)
                sd = _lane(dloc, l)
                for cc in range(D2 // 16):
                    hv = rows[j * 16 + l, pl.ds(cc * 16, 16)]
                    plsc.addupdate_scatter(
                        acc_v, [sd, cc * 16 + iota], hv * se)

    issue(jnp.int32(0), 0)

    def pair_body(p, _):
        ci0 = 2 * p
        issue(ci0 + 1, 1)
        wait(0)
        compute(ci0, 0)
        issue(ci0 + 2, 0)
        wait(1)
        compute(ci0 + 1, 1)
        return 0
    lax.fori_loop(0, nchunks // 2, pair_body, 0)
    wait(0)
    pltpu.sync_copy(acc_v, out_hbm.at[c, pl.ds(lo, RPT)])


_MESH = plsc.VectorSubcoreMesh(core_axis_name="c", subcore_axis_name="s")

_conv1_call = pl.kernel(
    _conv1_body,
    out_type=(
        jax.ShapeDtypeStruct((NHEADS, NROWS, D1), jnp.float32),
        jax.ShapeDtypeStruct((NT * CAP,), jnp.int32),
        jax.ShapeDtypeStruct((NT * 16,), jnp.int32),
    ),
    mesh=_MESH,
    compiler_params=pltpu.CompilerParams(
        needs_layout_passes=False, use_tc_tiling_on_sc=False),
    scratch_types=[
        pltpu.VMEM((SCAN,), jnp.int32),
        pltpu.VMEM((SCAN,), jnp.int32),
        pltpu.VMEM((CAP,), jnp.int32),
        pltpu.VMEM((CAP,), jnp.int32),
        pltpu.VMEM((W, D1), jnp.float32),
        pltpu.VMEM((RPT,), jnp.float32),
        pltpu.VMEM((RPT, D1), jnp.float32),
        pltpu.VMEM((16,), jnp.int32),
        pltpu.SMEM((NSB + 1,), jnp.int32),
        pltpu.SemaphoreType.DMA,
    ],
)

_conv2_call = pl.kernel(
    _conv2_body,
    out_type=jax.ShapeDtypeStruct((NC, NROWS, D2), jnp.float32),
    mesh=_MESH,
    compiler_params=pltpu.CompilerParams(
        needs_layout_passes=False, use_tc_tiling_on_sc=False),
    scratch_types=[
        pltpu.VMEM((KE2,), jnp.int32),
        pltpu.VMEM((KE2,), jnp.int32),
        pltpu.VMEM((KE2,), jnp.int32),
        pltpu.VMEM((KE2,), jnp.int32),
        pltpu.VMEM((KE2, D2), jnp.float32),
        pltpu.VMEM((KE2, D2), jnp.float32),
        pltpu.VMEM((RPT,), jnp.float32),
        pltpu.VMEM((RPT, D2), jnp.float32),
        pltpu.VMEM((16,), jnp.int32),
        pltpu.SemaphoreType.DMA,
        pltpu.SemaphoreType.DMA,
    ],
)


def _tail_body(g_ref, xt_ref, fc1_w_ref, fc1_b_ref, fc2_w_ref, fc2_b_ref,
               out_w_ref, out_b_ref, o_ref):
    xc = jnp.concatenate([g_ref[...], xt_ref[...]], axis=1)
    h1 = jnp.maximum(
        jnp.dot(xc, fc1_w_ref[...], preferred_element_type=jnp.float32)
        + fc1_b_ref[...][None, :], 0.0)
    h2 = jnp.maximum(
        jnp.dot(h1, fc2_w_ref[...], preferred_element_type=jnp.float32)
        + fc2_b_ref[...][None, :], 0.0)
    o_ref[...] = (
        jnp.dot(h2, out_w_ref[...], preferred_element_type=jnp.float32)
        + out_b_ref[...][None, :])


def _tail(g, xt, fc1_w, fc1_b, fc2_w, fc2_b, out_w, out_b):
    return pl.pallas_call(
        _tail_body,
        out_shape=jax.ShapeDtypeStruct((g.shape[0], 1), jnp.float32),
    )(g, xt, fc1_w, fc1_b, fc2_w, fc2_b, out_w, out_b)


def kernel(x, edge_index, batch, proteins, W1, a_src1, a_dst1, b1, W2, a_src2,
           a_dst2, b2, fc_g1_w, fc_g1_b, emb, conv_w, conv_b, fc_xt1_w,
           fc_xt1_b, fc1_w, fc1_b, fc2_w, fc2_b, out_w, out_b):
    loop = jnp.arange(N, dtype=edge_index.dtype)
    npad = EPAD - ETOT
    srcp = jnp.concatenate(
        [edge_index[0], loop, jnp.zeros((npad,), jnp.int32)])
    dstp = jnp.concatenate(
        [edge_index[1], loop, jnp.full((npad,), DSTPAD, jnp.int32)])

    h1 = x @ W1
    hr = h1.reshape(N, NHEADS, 78)
    alpha_src = jnp.einsum('nhc,hc->nh', hr, a_src1)
    alpha_dst = jnp.einsum('nhc,hc->nh', hr, a_dst1)
    adst_tab = jnp.pad(alpha_dst.T, ((0, 0), (0, NROWS - N)))
    ht1 = jnp.concatenate(
        [hr, jnp.ones((N, NHEADS, 1), jnp.float32),
         alpha_src[:, :, None]], axis=-1)
    ht1 = jnp.pad(ht1.transpose(1, 0, 2), ((0, 0), (0, NWIN - N), (0, 0)))
    zrow1 = jnp.zeros((RPT, D1), jnp.float32)

    out1, bkey, cnts = _conv1_call(srcp, dstp, adst_tab, ht1, zrow1)

    num1 = out1[:, :N, :78].transpose(1, 0, 2)
    den1 = out1[:, :N, 78].T[:, :, None]
    gat1 = (num1 / (den1 + 1e-16)).reshape(N, NHEADS * 78) + b1
    h2in = jax.nn.elu(gat1)

    h2 = h2in @ W2
    asrc2 = h2 @ a_src2[0]
    adst2 = jnp.pad(h2 @ a_dst2[0], (0, NROWS - N))
    ht2 = jnp.concatenate(
        [h2, jnp.ones((N, 1), jnp.float32), asrc2[:, None],
         jnp.zeros((N, D2 - 130), jnp.float32)], axis=1)
    zrow2 = jnp.zeros((RPT, D2), jnp.float32)

    out2 = _conv2_call(bkey, cnts, adst2, ht2, zrow2)
    o2 = out2[0, :N] + out2[1, :N]
    h3 = jax.nn.relu(o2[:, :128] / (o2[:, 128:129] + 1e-16) + b2)

    g = jax.ops.segment_max(h3, batch, num_segments=BATCH)
    g = jax.nn.relu(g @ fc_g1_w + fc_g1_b)

    e_xt = emb[proteins]
    conv = lax.conv_general_dilated(
        e_xt, conv_w, window_strides=(1,), padding='VALID',
        dimension_numbers=('NCH', 'OIH', 'NCH'))
    conv = jax.nn.relu(conv + conv_b[None, :, None])
    xt = conv.reshape(BATCH, 32 * 121) @ fc_xt1_w + fc_xt1_b

    return _tail(g, xt, fc1_w, fc1_b, fc2_w, fc2_b, out_w, out_b)


# speedup vs baseline: 9.8242x; 1.1381x over previous
import functools

import jax
import jax.numpy as jnp
from jax import lax
from jax.experimental import pallas as pl
from jax.experimental.pallas import tpu as pltpu
from jax.experimental.pallas import tpu_sc as plsc

N = 10000
BATCH = 256
NT = 16
NC = 2
NHEADS = 10
HPC = NHEADS // NC
D1 = 80
D2 = 144
RPT = 632
NROWS = RPT * NT
DSTPAD = 16000
CAP = 12288
SCAN = 2048
ETOT = 170000
EPAD = 172032
W = 512
NSB = 20
NWIN = NSB * W
KE2 = 128

_GDN = lax.GatherDimensionNumbers(
    offset_dims=(), collapsed_slice_dims=(0,), start_index_map=(0,))


def _lane(v, l):
    idx = jnp.full((16, 1), l, jnp.int32)
    return lax.gather(v, idx, _GDN, (1,),
                      mode=lax.GatherScatterMode.PROMISE_IN_BOUNDS)


def _conv1_body(src_hbm, dst_hbm, adst_hbm, ht_hbm, zrow_hbm,
                out_hbm, bkey_out, cnt_out, off_out,
                csrc_v, cdst_v, bkey_v, bkey2_v, win_v, adst_loc,
                acc_v, cntv_v, offv_v, sboff_s, semw):
    c = lax.axis_index("c")
    s = lax.axis_index("s")
    lo = s * RPT
    iota = lax.iota(jnp.int32, 16)

    def initb(i, _):
        bkey_v[pl.ds(i * 16, 16)] = jnp.zeros((16,), jnp.int32)
        bkey2_v[pl.ds(i * 16, 16)] = jnp.zeros((16,), jnp.int32)
        return 0
    lax.fori_loop(0, CAP // 16, initb, 0)

    def scan_chunk(ci, cnt):
        pltpu.sync_copy(src_hbm.at[pl.ds(ci * SCAN, SCAN)], csrc_v)
        pltpu.sync_copy(dst_hbm.at[pl.ds(ci * SCAN, SCAN)], cdst_v)

        def scan_vreg(j, cnt):
            sv = csrc_v[pl.ds(j * 16, 16)]
            dv = cdst_v[pl.ds(j * 16, 16)]
            m = (dv >= lo) & (dv < lo + RPT)
            mi = m.astype(jnp.int32)
            pos = jnp.minimum(cnt + plsc.cumsum(mi) - mi, CAP - 1)
            plsc.store_scatter(bkey_v, [pos], sv * 1024 + (dv - lo), mask=m)
            return jnp.minimum(cnt + jnp.sum(mi), CAP - 16)
        return lax.fori_loop(0, SCAN // 16, scan_vreg, cnt)
    cnt = lax.fori_loop(0, EPAD // SCAN, scan_chunk, jnp.int32(0))

    nv = (cnt + 15) // 16

    lane0 = lax.iota(jnp.int32, 16) == 0

    def part_sb(sb, off):
        sboff_s[sb] = off
        plsc.store_scatter(offv_v, [jnp.broadcast_to(sb, (16,))],
                           jnp.broadcast_to(off, (16,)), mask=lane0)

        def part_vreg(i, off):
            kv = bkey_v[pl.ds(i * 16, 16)]
            valid = (i * 16 + iota) < cnt
            m = (lax.shift_right_logical(kv, 19) == sb) & valid
            mi = m.astype(jnp.int32)
            tot = jnp.sum(mi)

            @pl.when(tot > 0)
            def _():
                pos = jnp.minimum(off + plsc.cumsum(mi) - mi, CAP - 1)
                plsc.store_scatter(bkey2_v, [pos], kv, mask=m)
            return off + tot
        return lax.fori_loop(0, nv, part_vreg, off)
    total = lax.fori_loop(0, NSB, part_sb, jnp.int32(0))
    sboff_s[NSB] = total
    plsc.store_scatter(offv_v, [jnp.broadcast_to(jnp.int32(NSB), (16,))],
                       jnp.broadcast_to(total, (16,)), mask=lane0)

    cntv_v[...] = jnp.broadcast_to(cnt, (16,))

    @pl.when(c == 0)
    def _publish():
        pltpu.sync_copy(bkey2_v, bkey_out.at[pl.ds(s * CAP, CAP)])
        pltpu.sync_copy(cntv_v, cnt_out.at[pl.ds(s * 16, 16)])
        pltpu.sync_copy(offv_v, off_out.at[pl.ds(s * 32, 32)])

    col79 = jnp.full((16,), 79, jnp.int32)

    def head_pass(k, _):
        h = c * HPC + k
        pltpu.sync_copy(adst_hbm.at[h, pl.ds(lo, RPT)], adst_loc)
        pltpu.sync_copy(zrow_hbm, acc_v)

        def sb_pass(sb, _):
            pltpu.async_copy(
                ht_hbm.at[h, pl.ds(sb * W, W)], win_v, semw).wait()
            sbstart = sboff_s[sb]
            sbcnt = sboff_s[sb + 1] - sbstart
            wbase = sb * W

            def edge_vreg(i, _):
                off = sbstart + i * 16
                kv = bkey2_v[pl.ds(off, 16)]
                srcw = jnp.clip(
                    lax.shift_right_logical(kv, 10) - wbase, 0, W - 1)
                dloc = kv & 1023
                av = plsc.load_gather(win_v, [srcw, col79])
                bv = plsc.load_gather(adst_loc, [dloc])
                ev = av + bv
                ev = jnp.where(ev >= 0.0, ev, 0.2 * ev)
                ee = jnp.exp(ev)
                ee = jnp.where(i * 16 + iota < sbcnt, ee, 0.0)

                @plsc.parallel_loop(0, 16, 1, unroll=8)
                def lp(l, ee=ee, dloc=dloc, srcw=srcw):
                    se = _lane(ee, l)
                    sd = _lane(dloc, l)
                    sw = _lane(srcw, l)
                    for cc in range(5):
                        hv = plsc.load_gather(win_v, [sw, cc * 16 + iota])
                        plsc.addupdate_scatter(
                            acc_v, [sd, cc * 16 + iota], hv * se)
                return 0
            lax.fori_loop(0, (sbcnt + 15) // 16, edge_vreg, 0)
            return 0
        lax.fori_loop(0, NSB, sb_pass, 0)
        pltpu.sync_copy(acc_v, out_hbm.at[h, pl.ds(lo, RPT)])
        return 0
    lax.fori_loop(0, HPC, head_pass, 0)


HROWS = 320
KBUF = 2048


def _conv2_body(bkey_hbm, off_hbm, adst_hbm, ht_hbm, zrow_hbm, out_hbm,
                kbuf, win_v, adst_loc, acc_v, offs_v, semw):
    c = lax.axis_index("c")
    s = lax.axis_index("s")
    lo = s * RPT
    iota = lax.iota(jnp.int32, 16)
    col129 = jnp.full((16,), 129, jnp.int32)

    pltpu.sync_copy(off_hbm.at[pl.ds(s * 32, 32)], offs_v)
    pltpu.sync_copy(zrow_hbm, acc_v)
    pltpu.sync_copy(adst_hbm.at[pl.ds(lo, RPT)], adst_loc)
    ov0 = offs_v[pl.ds(0, 16)]
    ov1 = offs_v[pl.ds(16, 16)]

    def off_at(sb):
        a = jnp.sum(jnp.where(iota == sb, ov0, 0))
        b = jnp.sum(jnp.where(iota == (sb - 16), ov1, 0))
        return jnp.where(sb < 16, a, b)

    minrow = c * 8
    shiftc = c * 312

    def sb_pass(sb, _):
        pltpu.async_copy(ht_hbm.at[pl.ds(sb * W, W)], win_v, semw).wait()
        sbstart = off_at(sb)
        sbcnt = off_at(sb + 1) - sbstart
        astart = jnp.minimum((sbstart // 8) * 8, CAP - KBUF)
        pltpu.sync_copy(bkey_hbm.at[pl.ds(s * CAP + astart, KBUF)], kbuf)
        abase = sbstart - astart
        wbase = sb * W

        def edge_vreg(i, _):
            kv = kbuf[pl.ds(abase + i * 16, 16)]
            srcw = jnp.clip(lax.shift_right_logical(kv, 10) - wbase, 0, W - 1)
            dloc = kv & 1023
            sd2 = dloc - shiftc
            valid = ((i * 16 + iota < sbcnt) & (sd2 >= minrow)
                     & (sd2 < HROWS))
            sd2 = jnp.clip(sd2, 0, HROWS - 1)
            av = plsc.load_gather(win_v, [srcw, col129])
            bv = plsc.load_gather(adst_loc, [dloc])
            ev = av + bv
            ev = jnp.where(ev >= 0.0, ev, 0.2 * ev)
            ee = jnp.exp(ev)
            ee = jnp.where(valid, ee, 0.0)

            @plsc.parallel_loop(0, 16, 1, unroll=8)
            def lp(l, ee=ee, sd2=sd2, srcw=srcw):
                se = _lane(ee, l)
                sd = _lane(sd2, l)
                sw = _lane(srcw, l)
                for cc in range(D2 // 16):
                    hv = plsc.load_gather(win_v, [sw, cc * 16 + iota])
                    plsc.addupdate_scatter(
                        acc_v, [sd, cc * 16 + iota], hv * se)
            return 0
        nvr = jnp.minimum((sbcnt + 15) // 16, (KBUF - 16) // 16)
        lax.fori_loop(0, nvr, edge_vreg, 0)
        return 0
    lax.fori_loop(0, NSB, sb_pass, 0)
    pltpu.sync_copy(acc_v, out_hbm.at[c, pl.ds(lo + c * 312, HROWS)])


_MESH = plsc.VectorSubcoreMesh(core_axis_name="c", subcore_axis_name="s")

_conv1_call = pl.kernel(
    _conv1_body,
    out_type=(
        jax.ShapeDtypeStruct((NHEADS, NROWS, D1), jnp.float32),
        jax.ShapeDtypeStruct((NT * CAP,), jnp.int32),
        jax.ShapeDtypeStruct((NT * 16,), jnp.int32),
        jax.ShapeDtypeStruct((NT * 32,), jnp.int32),
    ),
    mesh=_MESH,
    compiler_params=pltpu.CompilerParams(
        needs_layout_passes=False, use_tc_tiling_on_sc=False),
    scratch_types=[
        pltpu.VMEM((SCAN,), jnp.int32),
        pltpu.VMEM((SCAN,), jnp.int32),
        pltpu.VMEM((CAP,), jnp.int32),
        pltpu.VMEM((CAP,), jnp.int32),
        pltpu.VMEM((W, D1), jnp.float32),
        pltpu.VMEM((RPT,), jnp.float32),
        pltpu.VMEM((RPT, D1), jnp.float32),
        pltpu.VMEM((16,), jnp.int32),
        pltpu.VMEM((32,), jnp.int32),
        pltpu.SMEM((NSB + 1,), jnp.int32),
        pltpu.SemaphoreType.DMA,
    ],
)

_conv2_call = pl.kernel(
    _conv2_body,
    out_type=jax.ShapeDtypeStruct((NC, NROWS, D2), jnp.float32),
    mesh=_MESH,
    compiler_params=pltpu.CompilerParams(
        needs_layout_passes=False, use_tc_tiling_on_sc=False),
    scratch_types=[
        pltpu.VMEM((KBUF,), jnp.int32),
        pltpu.VMEM((W, D2), jnp.float32),
        pltpu.VMEM((RPT,), jnp.float32),
        pltpu.VMEM((HROWS, D2), jnp.float32),
        pltpu.VMEM((32,), jnp.int32),
        pltpu.SemaphoreType.DMA,
    ],
)


def _tail_body(g_ref, xt_ref, fc1_w_ref, fc1_b_ref, fc2_w_ref, fc2_b_ref,
               out_w_ref, out_b_ref, o_ref):
    xc = jnp.concatenate([g_ref[...], xt_ref[...]], axis=1)
    h1 = jnp.maximum(
        jnp.dot(xc, fc1_w_ref[...], preferred_element_type=jnp.float32)
        + fc1_b_ref[...][None, :], 0.0)
    h2 = jnp.maximum(
        jnp.dot(h1, fc2_w_ref[...], preferred_element_type=jnp.float32)
        + fc2_b_ref[...][None, :], 0.0)
    o_ref[...] = (
        jnp.dot(h2, out_w_ref[...], preferred_element_type=jnp.float32)
        + out_b_ref[...][None, :])


def _tail(g, xt, fc1_w, fc1_b, fc2_w, fc2_b, out_w, out_b):
    return pl.pallas_call(
        _tail_body,
        out_shape=jax.ShapeDtypeStruct((g.shape[0], 1), jnp.float32),
    )(g, xt, fc1_w, fc1_b, fc2_w, fc2_b, out_w, out_b)


def kernel(x, edge_index, batch, proteins, W1, a_src1, a_dst1, b1, W2, a_src2,
           a_dst2, b2, fc_g1_w, fc_g1_b, emb, conv_w, conv_b, fc_xt1_w,
           fc_xt1_b, fc1_w, fc1_b, fc2_w, fc2_b, out_w, out_b):
    loop = jnp.arange(N, dtype=edge_index.dtype)
    npad = EPAD - ETOT
    srcp = jnp.concatenate(
        [edge_index[0], loop, jnp.zeros((npad,), jnp.int32)])
    dstp = jnp.concatenate(
        [edge_index[1], loop, jnp.full((npad,), DSTPAD, jnp.int32)])

    h1 = x @ W1
    hr = h1.reshape(N, NHEADS, 78)
    alpha_src = jnp.einsum('nhc,hc->nh', hr, a_src1)
    alpha_dst = jnp.einsum('nhc,hc->nh', hr, a_dst1)
    adst_tab = jnp.pad(alpha_dst.T, ((0, 0), (0, NROWS - N)))
    ht1 = jnp.concatenate(
        [hr, jnp.ones((N, NHEADS, 1), jnp.float32),
         alpha_src[:, :, None]], axis=-1)
    ht1 = jnp.pad(ht1.transpose(1, 0, 2), ((0, 0), (0, NWIN - N), (0, 0)))
    zrow1 = jnp.zeros((RPT, D1), jnp.float32)

    out1, bkey, cnts, offs = _conv1_call(srcp, dstp, adst_tab, ht1, zrow1)

    num1 = out1[:, :N, :78].transpose(1, 0, 2)
    den1 = out1[:, :N, 78].T[:, :, None]
    gat1 = (num1 / (den1 + 1e-16)).reshape(N, NHEADS * 78) + b1
    h2in = jax.nn.elu(gat1)

    h2 = h2in @ W2
    asrc2 = h2 @ a_src2[0]
    adst2 = jnp.pad(h2 @ a_dst2[0], (0, NROWS - N))
    ht2 = jnp.concatenate(
        [h2, jnp.ones((N, 1), jnp.float32), asrc2[:, None],
         jnp.zeros((N, D2 - 130), jnp.float32)], axis=1)
    ht2 = jnp.pad(ht2, ((0, NWIN - N), (0, 0)))
    zrow2 = jnp.zeros((320, D2), jnp.float32)

    out2 = _conv2_call(bkey, offs, adst2, ht2, zrow2)
    r0 = out2[0].reshape(NT, RPT, D2)[:, :320]
    r1 = out2[1].reshape(NT, RPT, D2)[:, 320:]
    o2 = jnp.concatenate([r0, r1], axis=1).reshape(NROWS, D2)[:N]
    h3 = jax.nn.relu(o2[:, :128] / (o2[:, 128:129] + 1e-16) + b2)

    g = jax.ops.segment_max(h3, batch, num_segments=BATCH)
    g = jax.nn.relu(g @ fc_g1_w + fc_g1_b)

    e_xt = emb[proteins]
    conv = lax.conv_general_dilated(
        e_xt, conv_w, window_strides=(1,), padding='VALID',
        dimension_numbers=('NCH', 'OIH', 'NCH'))
    conv = jax.nn.relu(conv + conv_b[None, :, None])
    xt = conv.reshape(BATCH, 32 * 121) @ fc_xt1_w + fc_xt1_b

    return _tail(g, xt, fc1_w, fc1_b, fc2_w, fc2_b, out_w, out_b)
